# baseline (speedup 1.0000x reference)
import functools

import jax
import jax.numpy as jnp
from jax import lax
from jax.experimental import pallas as pl
from jax.experimental.pallas import tpu as pltpu
from jax.experimental.pallas import tpu_sc as plsc

_SC_INFO = plsc.get_sparse_core_info()
_NC = _SC_INFO.num_cores
_NS = _SC_INFO.num_subcores
_NW = _NC * _NS


def _sc_gather(table, idx):
    b = idx.shape[0]
    d = table.shape[1]
    b_per_w = b // _NW
    mesh = plsc.VectorSubcoreMesh(core_axis_name="c", subcore_axis_name="s")

    @functools.partial(
        pl.kernel,
        mesh=mesh,
        compiler_params=pltpu.CompilerParams(use_tc_tiling_on_sc=False),
        out_type=jax.ShapeDtypeStruct((b, d), jnp.float32),
        scratch_types=[
            pltpu.VMEM((b_per_w,), jnp.int32),
            pltpu.VMEM((b_per_w, d), jnp.float32),
            pltpu.SemaphoreType.DMA,
        ],
    )
    def k(table_hbm, idx_hbm, out_hbm, idx_v, rows_v, sem):
        wid = lax.axis_index("s") * _NC + lax.axis_index("c")
        base = wid * b_per_w
        pltpu.sync_copy(idx_hbm.at[pl.ds(base, b_per_w)], idx_v)
        pltpu.async_copy(table_hbm.at[idx_v], rows_v, sem).wait()
        pltpu.sync_copy(rows_v, out_hbm.at[pl.ds(base, b_per_w)])

    return k(table, idx)


def _tc_project(emb, W, v_blk=2048):
    bsz, d = emb.shape
    vocab = W.shape[0]
    grid = pl.cdiv(vocab, v_blk)

    def body(emb_ref, w_ref, out_ref):
        out_ref[...] = lax.dot_general(
            emb_ref[...], w_ref[...],
            dimension_numbers=(((1,), (1,)), ((), ())),
            preferred_element_type=jnp.float32,
        )

    return pl.pallas_call(
        body,
        grid=(grid,),
        in_specs=[
            pl.BlockSpec((bsz, d), lambda i: (0, 0)),
            pl.BlockSpec((v_blk, d), lambda i: (i, 0)),
        ],
        out_specs=pl.BlockSpec((bsz, v_blk), lambda i: (0, i)),
        out_shape=jax.ShapeDtypeStruct((bsz, vocab), jnp.float32),
    )(emb, W)


def kernel(x, tok_emb, W):
    emb = _sc_gather(tok_emb, x.astype(jnp.int32))
    return _tc_project(emb, W)


# speedup vs baseline: 2.8023x; 2.8023x over previous
import functools

import jax
import jax.numpy as jnp
from jax import lax
from jax.experimental import pallas as pl
from jax.experimental.pallas import tpu as pltpu
from jax.experimental.pallas import tpu_sc as plsc

_SC_INFO = plsc.get_sparse_core_info()
_NC = _SC_INFO.num_cores
_NS = _SC_INFO.num_subcores
_NW = _NC * _NS


def _sc_gather(table, idx):
    b = idx.shape[0]
    d = table.shape[1]
    b_per_w = b // _NW
    mesh = plsc.VectorSubcoreMesh(core_axis_name="c", subcore_axis_name="s")

    @functools.partial(
        pl.kernel,
        mesh=mesh,
        compiler_params=pltpu.CompilerParams(use_tc_tiling_on_sc=False),
        out_type=jax.ShapeDtypeStruct((b, d), jnp.float32),
        scratch_types=[
            pltpu.VMEM((b_per_w,), jnp.int32),
            pltpu.VMEM((b_per_w, d), jnp.float32),
            pltpu.SemaphoreType.DMA,
        ],
    )
    def k(table_hbm, idx_hbm, out_hbm, idx_v, rows_v, sem):
        wid = lax.axis_index("s") * _NC + lax.axis_index("c")
        base = wid * b_per_w
        pltpu.sync_copy(idx_hbm.at[pl.ds(base, b_per_w)], idx_v)
        pltpu.async_copy(table_hbm.at[idx_v], rows_v, sem).wait()
        pltpu.sync_copy(rows_v, out_hbm.at[pl.ds(base, b_per_w)])

    return k(table, idx)


def _tc_project_t(Wt, emb, v_blk=2048):
    d, vocab = Wt.shape
    bsz = emb.shape[0]
    grid = pl.cdiv(vocab, v_blk)

    def body(w_ref, emb_ref, out_ref):
        out_ref[...] = lax.dot_general(
            w_ref[...], emb_ref[...],
            dimension_numbers=(((0,), (1,)), ((), ())),
            preferred_element_type=jnp.float32,
        )

    return pl.pallas_call(
        body,
        grid=(grid,),
        in_specs=[
            pl.BlockSpec((d, v_blk), lambda i: (0, i)),
            pl.BlockSpec((bsz, d), lambda i: (0, 0)),
        ],
        out_specs=pl.BlockSpec((v_blk, bsz), lambda i: (i, 0)),
        out_shape=jax.ShapeDtypeStruct((vocab, bsz), jnp.float32),
    )(Wt, emb)


def kernel(x, tok_emb, W):
    emb = _sc_gather(tok_emb, x.astype(jnp.int32))
    out_t = _tc_project_t(W.T, emb)
    return out_t.T
